# baseline (speedup 1.0000x reference)
import jax
import jax.numpy as jnp
from jax import lax
from jax.experimental import pallas as pl
from jax.experimental.pallas import tpu as pltpu
from jax.experimental.pallas import tpu_sc as plsc

D_MODEL = 32
SCALE = float(D_MODEL) ** 0.5
VOCAB = 1000000

B, T = 4096, 200
PLANES_PER_SC = D_MODEL // 2
LOAD_TILES = 8
LOAD_CHUNK = VOCAB // LOAD_TILES

ROWS_HI = 13
ROWS_LO = 12
HI_TILES = T - 16 * ROWS_LO


def _body(xt_hbm, tt_hbm, out_hbm, idx_list, g0, g1, plane_sh,
          sgt, so0, so1):
    cid = lax.axis_index("c")
    sid = lax.axis_index("s")
    gath = (g0, g1)
    so = (so0, so1)

    hi = jnp.minimum(sid, HI_TILES)
    lo = jnp.maximum(sid - HI_TILES, 0)
    t0 = hi * ROWS_HI + lo * ROWS_LO
    nrows = jnp.where(sid < HI_TILES, ROWS_HI, ROWS_LO)

    for u in range(ROWS_HI):

        @pl.when(u < nrows)
        def _stage(u=u):
            pltpu.sync_copy(xt_hbm.at[t0 + u], idx_list[u])

    def gather_issue(u, jj):
        return pltpu.async_copy(plane_sh.at[idx_list[u]], gath[jj], sgt)

    def drain_outs():
        pltpu.make_async_copy(g0, out_hbm.at[0], so0).wait()
        pltpu.make_async_copy(g1, out_hbm.at[0], so1).wait()

    @pl.loop(0, PLANES_PER_SC)
    def _plane(p):
        plsc.subcore_barrier()
        cp = cid * PLANES_PER_SC + p

        @pl.when(sid < LOAD_TILES)
        def _load():
            off = sid * LOAD_CHUNK
            pltpu.sync_copy(tt_hbm.at[cp, pl.ds(off, LOAD_CHUNK)],
                            plane_sh.at[pl.ds(off, LOAD_CHUNK)])

        plsc.subcore_barrier()

        @pl.when(p > 0)
        def _drain():
            drain_outs()

        gather_issue(0, 0)
        for u in range(ROWS_HI):
            jj = u % 2
            if u + 1 < ROWS_HI:
                nxt = (u + 1) % 2

                @pl.when(u + 1 < nrows)
                def _issue(u=u, nxt=nxt):
                    if u + 1 >= 2:
                        pltpu.make_async_copy(
                            gath[nxt], out_hbm.at[0], so[nxt]).wait()
                    gather_issue(u + 1, nxt)

            @pl.when(u < nrows)
            def _process(u=u, jj=jj, cp=cp):
                pltpu.make_async_copy(
                    plane_sh.at[idx_list[u]], gath[jj], sgt).wait()
                gj = gath[jj]

                @pl.loop(0, B // 128)
                def _scale(i):
                    for k in range(8):
                        o = i * 128 + k * 16
                        gj[pl.ds(o, 16)] = gj[pl.ds(o, 16)] * SCALE

                pltpu.async_copy(
                    gj, out_hbm.at[(t0 + u) * D_MODEL + cp], so[jj])

    drain_outs()


def kernel(x, table):
    xt = jnp.transpose(x).astype(jnp.int32)
    tt = jnp.transpose(table)
    mesh = plsc.VectorSubcoreMesh(core_axis_name="c", subcore_axis_name="s")
    out = pl.kernel(
        _body,
        out_type=jax.ShapeDtypeStruct((T * D_MODEL, B), jnp.float32),
        mesh=mesh,
        compiler_params=pltpu.CompilerParams(use_tc_tiling_on_sc=False),
        scratch_types=(
            [pltpu.VMEM((B,), jnp.int32) for _ in range(ROWS_HI)],
            pltpu.VMEM((B,), jnp.float32),
            pltpu.VMEM((B,), jnp.float32),
            pltpu.VMEM_SHARED((VOCAB,), jnp.float32),
            pltpu.SemaphoreType.DMA,
            pltpu.SemaphoreType.DMA,
            pltpu.SemaphoreType.DMA,
        ),
    )(xt, tt)
    out = out.reshape(T, D_MODEL, B)
    return jnp.transpose(out, (2, 0, 1))


# speedup vs baseline: 3.0374x; 3.0374x over previous
import jax
import jax.numpy as jnp
from jax import lax
from jax.experimental import pallas as pl
from jax.experimental.pallas import tpu as pltpu
from jax.experimental.pallas import tpu_sc as plsc

D_MODEL = 32
SCALE = float(D_MODEL) ** 0.5

B, T = 4096, 200
N = B * T
NUM_WORKERS = 32
PER_W = N // NUM_WORKERS
NBUF = 4
CHUNK = 800
NCHUNK = PER_W // CHUNK
NBLK = NCHUNK // NBUF


def _body(x_hbm, table_hbm, out_hbm, idx_all,
          r0, r1, r2, r3, g0, g1, g2, g3, o0, o1, o2, o3):
    rows = (r0, r1, r2, r3)
    sg = (g0, g1, g2, g3)
    so = (o0, o1, o2, o3)

    wid = lax.axis_index("s") * 2 + lax.axis_index("c")
    base = wid * PER_W
    pltpu.sync_copy(x_hbm.at[pl.ds(base, PER_W)], idx_all)

    def gather(c, b):
        return pltpu.async_copy(
            table_hbm.at[idx_all.at[pl.ds(c * CHUNK, CHUNK)]], rows[b], sg[b])

    def write_out(c, b):
        return pltpu.async_copy(
            rows[b], out_hbm.at[pl.ds(base + c * CHUNK, CHUNK)], so[b])

    gd = {}
    od = {}
    for b in range(NBUF):
        gd[b] = gather(b, b)

    for blk in range(NBLK):
        for b in range(NBUF):
            c = blk * NBUF + b
            gd[b].wait()
            rb = rows[b]

            @pl.loop(0, CHUNK, unroll=8)
            def _scale(j):
                rb[j, 0:16] = rb[j, 0:16] * SCALE
                rb[j, 16:32] = rb[j, 16:32] * SCALE

            od[b] = write_out(c, b)
        for b in range(NBUF):
            cn = blk * NBUF + b + NBUF
            if cn < NCHUNK:
                od[b].wait()
                gd[b] = gather(cn, b)

    for b in range(NBUF):
        od[b].wait()


def kernel(x, table):
    xf = x.reshape(N).astype(jnp.int32)
    mesh = plsc.VectorSubcoreMesh(core_axis_name="c", subcore_axis_name="s")
    out = pl.kernel(
        _body,
        out_type=jax.ShapeDtypeStruct((N, D_MODEL), jnp.float32),
        mesh=mesh,
        compiler_params=pltpu.CompilerParams(use_tc_tiling_on_sc=False),
        scratch_types=(
            [pltpu.VMEM((PER_W,), jnp.int32)]
            + [pltpu.VMEM((CHUNK, D_MODEL), jnp.float32) for _ in range(NBUF)]
            + [pltpu.SemaphoreType.DMA for _ in range(2 * NBUF)]
        ),
    )(xf, table)
    return out.reshape(B, T, D_MODEL)
